# baseline (speedup 1.0000x reference)
import functools

import jax
import jax.numpy as jnp
from jax import lax
from jax.experimental import pallas as pl
from jax.experimental.pallas import tpu as pltpu
from jax.experimental.pallas import tpu_sc as plsc

N = 10000
D = 128
E = 320000

NC = 2
NS = 16
NW = NC * NS

N_PAD = 10240
ROWS_PER_TILE = N_PAD // NS
CH = 128
E_PER_TILE = 10240
E_PAD = NW * E_PER_TILE
N_CH = E_PER_TILE // CH

BLK = 1024
GRID = N_PAD // BLK


def _dotT(a, w):
    return lax.dot_general(a, w, (((1,), (1,)), ((), ())),
                           precision=lax.Precision.HIGHEST,
                           preferred_element_type=jnp.float32)


def _proj_body(x_ref, w_ref, b_ref, o_ref):
    o_ref[...] = jnp.maximum(_dotT(x_ref[...], w_ref[...]) + b_ref[...], 0.0)


def _tc_proj(x, w, b):
    return pl.pallas_call(
        _proj_body,
        grid=(GRID,),
        in_specs=[
            pl.BlockSpec((BLK, D), lambda i: (i, 0)),
            pl.BlockSpec((D, D), lambda i: (0, 0)),
            pl.BlockSpec((1, D), lambda i: (0, 0)),
        ],
        out_specs=pl.BlockSpec((BLK, D), lambda i: (i, 0)),
        out_shape=jax.ShapeDtypeStruct((N_PAD, D), jnp.float32),
    )(x, w, b.reshape(1, D))


def _combine(parts_ref, cnt_ref, x_ref, wl_ref, bl_ref, wr_ref, g_ref, b_ref):
    p = parts_ref[0] + parts_ref[1]
    c = jnp.sum(cnt_ref[...], axis=0)[:, None]
    agg = p / jnp.maximum(c, 1.0)
    out = _dotT(agg, wl_ref[...]) + bl_ref[...] + _dotT(x_ref[...], wr_ref[...])
    mu = jnp.mean(out, axis=-1, keepdims=True)
    d = out - mu
    var = jnp.mean(d * d, axis=-1, keepdims=True)
    return d / jnp.sqrt(var + 1e-5) * g_ref[...] + b_ref[...]


def _mid_body(parts_ref, cnt_ref, x_ref, wl_ref, bl_ref, wr_ref, g_ref, b_ref,
              wp_ref, bp_ref, y_ref, h_ref):
    y = _combine(parts_ref, cnt_ref, x_ref, wl_ref, bl_ref, wr_ref, g_ref, b_ref)
    y_ref[...] = y
    h_ref[...] = jnp.maximum(_dotT(y, wp_ref[...]) + bp_ref[...], 0.0)


def _final_body(parts_ref, cnt_ref, x_ref, wl_ref, bl_ref, wr_ref, g_ref,
                b_ref, y_ref):
    y_ref[...] = _combine(parts_ref, cnt_ref, x_ref, wl_ref, bl_ref, wr_ref,
                          g_ref, b_ref)


_W_SPEC = pl.BlockSpec((D, D), lambda i: (0, 0))
_V_SPEC = pl.BlockSpec((1, D), lambda i: (0, 0))
_ROW_SPEC = pl.BlockSpec((BLK, D), lambda i: (i, 0))
_PARTS_SPEC = pl.BlockSpec((NC, BLK, D), lambda i: (0, i, 0))
_CNT_SPEC = pl.BlockSpec((NW, BLK), lambda i: (0, i))


def _tc_mid(parts, cnt, x, wl, bl, wr, g, b, wp, bp):
    return pl.pallas_call(
        _mid_body,
        grid=(GRID,),
        in_specs=[_PARTS_SPEC, _CNT_SPEC, _ROW_SPEC, _W_SPEC, _V_SPEC,
                  _W_SPEC, _V_SPEC, _V_SPEC, _W_SPEC, _V_SPEC],
        out_specs=(_ROW_SPEC, _ROW_SPEC),
        out_shape=(jax.ShapeDtypeStruct((N_PAD, D), jnp.float32),
                   jax.ShapeDtypeStruct((N_PAD, D), jnp.float32)),
    )(parts, cnt, x, wl, bl.reshape(1, D), wr, g.reshape(1, D),
      b.reshape(1, D), wp, bp.reshape(1, D))


def _tc_final(parts, cnt, x, wl, bl, wr, g, b):
    return pl.pallas_call(
        _final_body,
        grid=(GRID,),
        in_specs=[_PARTS_SPEC, _CNT_SPEC, _ROW_SPEC, _W_SPEC, _V_SPEC,
                  _W_SPEC, _V_SPEC, _V_SPEC],
        out_specs=_ROW_SPEC,
        out_shape=jax.ShapeDtypeStruct((N_PAD, D), jnp.float32),
    )(parts, cnt, x, wl, bl.reshape(1, D), wr, g.reshape(1, D),
      b.reshape(1, D))


_MESH = plsc.VectorSubcoreMesh(core_axis_name="c", subcore_axis_name="s",
                               num_cores=NC, num_subcores=NS)


GRP = 8
NB = 2
N_GRP_TOTAL = E_PAD // (NC * NS * GRP * CH)
K0 = 6
K1 = 2 * N_GRP_TOTAL - K0


def _tile_layout(c, s):
    n_grp = jnp.where(c == 0, K0, K1)
    row_base = jnp.where(c == 0, s * (K0 * GRP),
                         16 * (K0 * GRP) + s * (K1 * GRP))
    return n_grp, row_base


def _edge_group(i, row_base, h_hbm, src2d_hbm, dst2d_hbm, src_g, dst_g, rows,
                agg_sh):
    row0 = pl.multiple_of(row_base + i * GRP, 8)
    pltpu.sync_copy(src2d_hbm.at[pl.ds(row0, GRP)], src_g)
    pltpu.sync_copy(dst2d_hbm.at[pl.ds(row0, GRP)], dst_g)

    for k in range(GRP):
        b = k % NB
        pltpu.sync_copy(h_hbm.at[src_g.at[k]], rows[b])
        pltpu.sync_copy(rows[b], agg_sh.at[dst_g.at[k]], add=True)


def _sc_body_cnt(h_hbm, src2d_hbm, dst2d_hbm, zrows_hbm, zcnt_hbm,
                 parts_out, cnt_out, src_g, dst_g, r0, r1,
                 cnt_v, agg_sh):
    rows = (r0, r1)
    c = lax.axis_index("c")
    s = lax.axis_index("s")
    wid = s * NC + c
    rbase = s * ROWS_PER_TILE
    pltpu.sync_copy(zrows_hbm.at[pl.ds(rbase, ROWS_PER_TILE)],
                    agg_sh.at[pl.ds(rbase, ROWS_PER_TILE)])
    pltpu.sync_copy(zcnt_hbm, cnt_v)
    plsc.subcore_barrier()

    ones16 = jnp.full((16,), 1.0, jnp.float32)
    n_grp, row_base = _tile_layout(c, s)

    @pl.loop(0, n_grp)
    def body(i):
        _edge_group(i, row_base, h_hbm, src2d_hbm, dst2d_hbm, src_g, dst_g,
                    rows, agg_sh)
        for k in range(GRP):
            for j in range(CH // 16):
                plsc.addupdate_scatter(
                    cnt_v, [dst_g[k, pl.ds(j * 16, 16)]], ones16)

    plsc.subcore_barrier()
    pltpu.sync_copy(agg_sh.at[pl.ds(rbase, ROWS_PER_TILE)],
                    parts_out.at[c, pl.ds(rbase, ROWS_PER_TILE)])
    pltpu.sync_copy(cnt_v, cnt_out.at[wid])


def _sc_body_nocnt(h_hbm, src2d_hbm, dst2d_hbm, zrows_hbm, parts_out, src_g,
                   dst_g, r0, r1, agg_sh):
    rows = (r0, r1)
    c = lax.axis_index("c")
    s = lax.axis_index("s")
    wid = s * NC + c
    rbase = s * ROWS_PER_TILE
    pltpu.sync_copy(zrows_hbm.at[pl.ds(rbase, ROWS_PER_TILE)],
                    agg_sh.at[pl.ds(rbase, ROWS_PER_TILE)])
    plsc.subcore_barrier()

    n_grp, row_base = _tile_layout(c, s)

    @pl.loop(0, n_grp)
    def body(i):
        _edge_group(i, row_base, h_hbm, src2d_hbm, dst2d_hbm, src_g, dst_g,
                    rows, agg_sh)

    plsc.subcore_barrier()
    pltpu.sync_copy(agg_sh.at[pl.ds(rbase, ROWS_PER_TILE)],
                    parts_out.at[c, pl.ds(rbase, ROWS_PER_TILE)])


_sc_edges_cnt = functools.partial(
    pl.kernel,
    _sc_body_cnt,
    out_type=(jax.ShapeDtypeStruct((NC, N_PAD, D), jnp.float32),
              jax.ShapeDtypeStruct((NW, N_PAD), jnp.float32)),
    mesh=_MESH,
    scratch_types=[
        pltpu.VMEM((GRP, CH), jnp.int32),
        pltpu.VMEM((GRP, CH), jnp.int32),
        pltpu.VMEM((CH, D), jnp.float32),
        pltpu.VMEM((CH, D), jnp.float32),
        pltpu.VMEM((N_PAD,), jnp.float32),
        pltpu.VMEM_SHARED((N_PAD, D), jnp.float32),
    ],
    compiler_params=pltpu.CompilerParams(needs_layout_passes=False),
)()

_sc_edges_nocnt = functools.partial(
    pl.kernel,
    _sc_body_nocnt,
    out_type=jax.ShapeDtypeStruct((NC, N_PAD, D), jnp.float32),
    mesh=_MESH,
    scratch_types=[
        pltpu.VMEM((GRP, CH), jnp.int32),
        pltpu.VMEM((GRP, CH), jnp.int32),
        pltpu.VMEM((CH, D), jnp.float32),
        pltpu.VMEM((CH, D), jnp.float32),
        pltpu.VMEM_SHARED((N_PAD, D), jnp.float32),
    ],
)()


@jax.jit
def kernel(x, edge_index, Wp0, bp0, Wl0, bl0, Wr0, g0, b0,
           Wp1, bp1, Wl1, bl1, Wr1, g1, b1):
    x_pad = jnp.pad(x, ((0, N_PAD - N), (0, 0)))
    src = jnp.pad(edge_index[0], (0, E_PAD - E))
    dst = jnp.pad(edge_index[1], (0, E_PAD - E), constant_values=N_PAD - 1)
    zrows = jnp.zeros((N_PAD, D), jnp.float32)
    zcnt = jnp.zeros((N_PAD,), jnp.float32)

    src2d = src.reshape(E_PAD // CH, CH)
    dst2d = dst.reshape(E_PAD // CH, CH)

    h0p = _tc_proj(x_pad, Wp0, bp0)
    parts0, cnt = _sc_edges_cnt(h0p, src2d, dst2d, zrows, zcnt)
    h0, h1p = _tc_mid(parts0, cnt, x_pad, Wl0, bl0, Wr0, g0, b0, Wp1, bp1)
    parts1 = _sc_edges_nocnt(h1p, src2d, dst2d, zrows)
    h1 = _tc_final(parts1, cnt, h0, Wl1, bl1, Wr1, g1, b1)
    return h1[:N]


# speedup vs baseline: 1.2623x; 1.2623x over previous
import functools

import jax
import jax.numpy as jnp
from jax import lax
from jax.experimental import pallas as pl
from jax.experimental.pallas import tpu as pltpu
from jax.experimental.pallas import tpu_sc as plsc

N = 10000
D = 128
E = 320000

NC = 2
NS = 16
NW = NC * NS

N_PAD = 10240
ROWS_PER_TILE = N_PAD // NS
CH = 128
E_PER_TILE = 10240
E_PAD = NW * E_PER_TILE
N_CH = E_PER_TILE // CH

BLK = 1024
GRID = N_PAD // BLK


def _dotT(a, w):
    return lax.dot_general(a, w, (((1,), (1,)), ((), ())),
                           precision=lax.Precision.HIGHEST,
                           preferred_element_type=jnp.float32)


def _proj_body(x_ref, w_ref, b_ref, o_ref):
    o_ref[...] = jnp.maximum(_dotT(x_ref[...], w_ref[...]) + b_ref[...], 0.0)


def _tc_proj(x, w, b):
    return pl.pallas_call(
        _proj_body,
        grid=(GRID,),
        in_specs=[
            pl.BlockSpec((BLK, D), lambda i: (i, 0)),
            pl.BlockSpec((D, D), lambda i: (0, 0)),
            pl.BlockSpec((1, D), lambda i: (0, 0)),
        ],
        out_specs=pl.BlockSpec((BLK, D), lambda i: (i, 0)),
        out_shape=jax.ShapeDtypeStruct((N_PAD, D), jnp.float32),
    )(x, w, b.reshape(1, D))


def _combine(parts_ref, cnt_ref, x_ref, wl_ref, bl_ref, wr_ref, g_ref, b_ref):
    p = parts_ref[0] + parts_ref[1]
    c = jnp.sum(cnt_ref[...], axis=0)[:, None]
    agg = p / jnp.maximum(c, 1.0)
    out = _dotT(agg, wl_ref[...]) + bl_ref[...] + _dotT(x_ref[...], wr_ref[...])
    mu = jnp.mean(out, axis=-1, keepdims=True)
    d = out - mu
    var = jnp.mean(d * d, axis=-1, keepdims=True)
    return d / jnp.sqrt(var + 1e-5) * g_ref[...] + b_ref[...]


def _mid_body(parts_ref, cnt_ref, x_ref, wl_ref, bl_ref, wr_ref, g_ref, b_ref,
              wp_ref, bp_ref, y_ref, h_ref):
    y = _combine(parts_ref, cnt_ref, x_ref, wl_ref, bl_ref, wr_ref, g_ref, b_ref)
    y_ref[...] = y
    h_ref[...] = jnp.maximum(_dotT(y, wp_ref[...]) + bp_ref[...], 0.0)


def _final_body(parts_ref, cnt_ref, x_ref, wl_ref, bl_ref, wr_ref, g_ref,
                b_ref, y_ref):
    y_ref[...] = _combine(parts_ref, cnt_ref, x_ref, wl_ref, bl_ref, wr_ref,
                          g_ref, b_ref)


_W_SPEC = pl.BlockSpec((D, D), lambda i: (0, 0))
_V_SPEC = pl.BlockSpec((1, D), lambda i: (0, 0))
_ROW_SPEC = pl.BlockSpec((BLK, D), lambda i: (i, 0))
_PARTS_SPEC = pl.BlockSpec((NC, BLK, D), lambda i: (0, i, 0))
_CNT_SPEC = pl.BlockSpec((NW, BLK), lambda i: (0, i))


def _tc_mid(parts, cnt, x, wl, bl, wr, g, b, wp, bp):
    return pl.pallas_call(
        _mid_body,
        grid=(GRID,),
        in_specs=[_PARTS_SPEC, _CNT_SPEC, _ROW_SPEC, _W_SPEC, _V_SPEC,
                  _W_SPEC, _V_SPEC, _V_SPEC, _W_SPEC, _V_SPEC],
        out_specs=(_ROW_SPEC, _ROW_SPEC),
        out_shape=(jax.ShapeDtypeStruct((N_PAD, D), jnp.float32),
                   jax.ShapeDtypeStruct((N_PAD, D), jnp.float32)),
    )(parts, cnt, x, wl, bl.reshape(1, D), wr, g.reshape(1, D),
      b.reshape(1, D), wp, bp.reshape(1, D))


def _tc_final(parts, cnt, x, wl, bl, wr, g, b):
    return pl.pallas_call(
        _final_body,
        grid=(GRID,),
        in_specs=[_PARTS_SPEC, _CNT_SPEC, _ROW_SPEC, _W_SPEC, _V_SPEC,
                  _W_SPEC, _V_SPEC, _V_SPEC],
        out_specs=_ROW_SPEC,
        out_shape=jax.ShapeDtypeStruct((N_PAD, D), jnp.float32),
    )(parts, cnt, x, wl, bl.reshape(1, D), wr, g.reshape(1, D),
      b.reshape(1, D))


_MESH = plsc.VectorSubcoreMesh(core_axis_name="c", subcore_axis_name="s",
                               num_cores=NC, num_subcores=NS)


GRP = 8
NB = 2
N_GRP_TOTAL = E_PAD // (NC * NS * GRP * CH)
K0 = 14
K1 = 2 * N_GRP_TOTAL - K0


def _tile_layout(c, s):
    n_grp = jnp.where(c == 0, K0, K1)
    row_base = jnp.where(c == 0, s * (K0 * GRP),
                         16 * (K0 * GRP) + s * (K1 * GRP))
    return n_grp, row_base


def _edge_group(i, row_base, h_hbm, src2d_hbm, dst2d_hbm, src_g, dst_g, rows,
                agg_sh):
    row0 = pl.multiple_of(row_base + i * GRP, 8)
    pltpu.sync_copy(src2d_hbm.at[pl.ds(row0, GRP)], src_g)
    pltpu.sync_copy(dst2d_hbm.at[pl.ds(row0, GRP)], dst_g)

    for k in range(GRP):
        b = k % NB
        pltpu.sync_copy(h_hbm.at[src_g.at[k]], rows[b])
        pltpu.sync_copy(rows[b], agg_sh.at[dst_g.at[k]], add=True)


def _sc_body_cnt(h_hbm, src2d_hbm, dst2d_hbm, zrows_hbm, zcnt_hbm,
                 parts_out, cnt_out, src_g, dst_g, r0, r1,
                 cnt_v, agg_sh):
    rows = (r0, r1)
    c = lax.axis_index("c")
    s = lax.axis_index("s")
    wid = s * NC + c
    rbase = s * ROWS_PER_TILE
    pltpu.sync_copy(zrows_hbm.at[pl.ds(rbase, ROWS_PER_TILE)],
                    agg_sh.at[pl.ds(rbase, ROWS_PER_TILE)])
    pltpu.sync_copy(zcnt_hbm, cnt_v)
    plsc.subcore_barrier()

    ones16 = jnp.full((16,), 1.0, jnp.float32)
    n_grp, row_base = _tile_layout(c, s)

    @pl.loop(0, n_grp)
    def body(i):
        _edge_group(i, row_base, h_hbm, src2d_hbm, dst2d_hbm, src_g, dst_g,
                    rows, agg_sh)
        for k in range(GRP):
            for j in range(CH // 16):
                plsc.addupdate_scatter(
                    cnt_v, [dst_g[k, pl.ds(j * 16, 16)]], ones16)

    plsc.subcore_barrier()
    pltpu.sync_copy(agg_sh.at[pl.ds(rbase, ROWS_PER_TILE)],
                    parts_out.at[c, pl.ds(rbase, ROWS_PER_TILE)])
    pltpu.sync_copy(cnt_v, cnt_out.at[wid])


def _sc_body_nocnt(h_hbm, src2d_hbm, dst2d_hbm, zrows_hbm, parts_out, src_g,
                   dst_g, r0, r1, agg_sh):
    rows = (r0, r1)
    c = lax.axis_index("c")
    s = lax.axis_index("s")
    wid = s * NC + c
    rbase = s * ROWS_PER_TILE
    pltpu.sync_copy(zrows_hbm.at[pl.ds(rbase, ROWS_PER_TILE)],
                    agg_sh.at[pl.ds(rbase, ROWS_PER_TILE)])
    plsc.subcore_barrier()

    n_grp, row_base = _tile_layout(c, s)

    @pl.loop(0, n_grp)
    def body(i):
        _edge_group(i, row_base, h_hbm, src2d_hbm, dst2d_hbm, src_g, dst_g,
                    rows, agg_sh)

    plsc.subcore_barrier()
    pltpu.sync_copy(agg_sh.at[pl.ds(rbase, ROWS_PER_TILE)],
                    parts_out.at[c, pl.ds(rbase, ROWS_PER_TILE)])


_sc_edges_cnt = functools.partial(
    pl.kernel,
    _sc_body_cnt,
    out_type=(jax.ShapeDtypeStruct((NC, N_PAD, D), jnp.float32),
              jax.ShapeDtypeStruct((NW, N_PAD), jnp.float32)),
    mesh=_MESH,
    scratch_types=[
        pltpu.VMEM((GRP, CH), jnp.int32),
        pltpu.VMEM((GRP, CH), jnp.int32),
        pltpu.VMEM((CH, D), jnp.float32),
        pltpu.VMEM((CH, D), jnp.float32),
        pltpu.VMEM((N_PAD,), jnp.float32),
        pltpu.VMEM_SHARED((N_PAD, D), jnp.float32),
    ],
    compiler_params=pltpu.CompilerParams(needs_layout_passes=False),
)()

_sc_edges_nocnt = functools.partial(
    pl.kernel,
    _sc_body_nocnt,
    out_type=jax.ShapeDtypeStruct((NC, N_PAD, D), jnp.float32),
    mesh=_MESH,
    scratch_types=[
        pltpu.VMEM((GRP, CH), jnp.int32),
        pltpu.VMEM((GRP, CH), jnp.int32),
        pltpu.VMEM((CH, D), jnp.float32),
        pltpu.VMEM((CH, D), jnp.float32),
        pltpu.VMEM_SHARED((N_PAD, D), jnp.float32),
    ],
)()


@jax.jit
def kernel(x, edge_index, Wp0, bp0, Wl0, bl0, Wr0, g0, b0,
           Wp1, bp1, Wl1, bl1, Wr1, g1, b1):
    x_pad = jnp.pad(x, ((0, N_PAD - N), (0, 0)))
    src = jnp.pad(edge_index[0], (0, E_PAD - E))
    dst = jnp.pad(edge_index[1], (0, E_PAD - E), constant_values=N_PAD - 1)
    zrows = jnp.zeros((N_PAD, D), jnp.float32)
    zcnt = jnp.zeros((N_PAD,), jnp.float32)

    src2d = src.reshape(E_PAD // CH, CH)
    dst2d = dst.reshape(E_PAD // CH, CH)

    h0p = _tc_proj(x_pad, Wp0, bp0)
    parts0, cnt = _sc_edges_cnt(h0p, src2d, dst2d, zrows, zcnt)
    h0, h1p = _tc_mid(parts0, cnt, x_pad, Wl0, bl0, Wr0, g0, b0, Wp1, bp1)
    parts1 = _sc_edges_nocnt(h1p, src2d, dst2d, zrows)
    h1 = _tc_final(parts1, cnt, h0, Wl1, bl1, Wr1, g1, b1)
    return h1[:N]


# speedup vs baseline: 1.3162x; 1.0427x over previous
import functools

import jax
import jax.numpy as jnp
from jax import lax
from jax.experimental import pallas as pl
from jax.experimental.pallas import tpu as pltpu
from jax.experimental.pallas import tpu_sc as plsc

N = 10000
D = 128
E = 320000

NC = 2
NS = 16
NW = NC * NS

N_PAD = 10240
ROWS_PER_TILE = N_PAD // NS
CH = 128
E_PER_TILE = 10240
E_PAD = NW * E_PER_TILE
N_CH = E_PER_TILE // CH

BLK = 1024
GRID = N_PAD // BLK


def _dotT(a, w):
    return lax.dot_general(a, w, (((1,), (1,)), ((), ())),
                           precision=lax.Precision.HIGHEST,
                           preferred_element_type=jnp.float32)


def _proj_body(x_ref, w_ref, b_ref, o_ref):
    o_ref[...] = jnp.maximum(_dotT(x_ref[...], w_ref[...]) + b_ref[...], 0.0)


def _tc_proj(x, w, b):
    return pl.pallas_call(
        _proj_body,
        grid=(GRID,),
        in_specs=[
            pl.BlockSpec((BLK, D), lambda i: (i, 0)),
            pl.BlockSpec((D, D), lambda i: (0, 0)),
            pl.BlockSpec((1, D), lambda i: (0, 0)),
        ],
        out_specs=pl.BlockSpec((BLK, D), lambda i: (i, 0)),
        out_shape=jax.ShapeDtypeStruct((N_PAD, D), jnp.float32),
    )(x, w, b.reshape(1, D))


def _combine(parts_ref, cnt_ref, x_ref, wl_ref, bl_ref, wr_ref, g_ref, b_ref):
    p = parts_ref[0] + parts_ref[1]
    c = jnp.sum(cnt_ref[...], axis=0)[:, None]
    agg = p / jnp.maximum(c, 1.0)
    out = _dotT(agg, wl_ref[...]) + bl_ref[...] + _dotT(x_ref[...], wr_ref[...])
    mu = jnp.mean(out, axis=-1, keepdims=True)
    d = out - mu
    var = jnp.mean(d * d, axis=-1, keepdims=True)
    return d / jnp.sqrt(var + 1e-5) * g_ref[...] + b_ref[...]


def _mid_body(parts_ref, cnt_ref, x_ref, wl_ref, bl_ref, wr_ref, g_ref, b_ref,
              wp_ref, bp_ref, y_ref, h_ref):
    y = _combine(parts_ref, cnt_ref, x_ref, wl_ref, bl_ref, wr_ref, g_ref, b_ref)
    y_ref[...] = y
    h_ref[...] = jnp.maximum(_dotT(y, wp_ref[...]) + bp_ref[...], 0.0)


def _final_body(parts_ref, cnt_ref, x_ref, wl_ref, bl_ref, wr_ref, g_ref,
                b_ref, y_ref):
    y_ref[...] = _combine(parts_ref, cnt_ref, x_ref, wl_ref, bl_ref, wr_ref,
                          g_ref, b_ref)


_W_SPEC = pl.BlockSpec((D, D), lambda i: (0, 0))
_V_SPEC = pl.BlockSpec((1, D), lambda i: (0, 0))
_ROW_SPEC = pl.BlockSpec((BLK, D), lambda i: (i, 0))
_PARTS_SPEC = pl.BlockSpec((NC, BLK, D), lambda i: (0, i, 0))
_CNT_SPEC = pl.BlockSpec((NW, BLK), lambda i: (0, i))


def _tc_mid(parts, cnt, x, wl, bl, wr, g, b, wp, bp):
    return pl.pallas_call(
        _mid_body,
        grid=(GRID,),
        in_specs=[_PARTS_SPEC, _CNT_SPEC, _ROW_SPEC, _W_SPEC, _V_SPEC,
                  _W_SPEC, _V_SPEC, _V_SPEC, _W_SPEC, _V_SPEC],
        out_specs=(_ROW_SPEC, _ROW_SPEC),
        out_shape=(jax.ShapeDtypeStruct((N_PAD, D), jnp.float32),
                   jax.ShapeDtypeStruct((N_PAD, D), jnp.float32)),
    )(parts, cnt, x, wl, bl.reshape(1, D), wr, g.reshape(1, D),
      b.reshape(1, D), wp, bp.reshape(1, D))


def _tc_final(parts, cnt, x, wl, bl, wr, g, b):
    return pl.pallas_call(
        _final_body,
        grid=(GRID,),
        in_specs=[_PARTS_SPEC, _CNT_SPEC, _ROW_SPEC, _W_SPEC, _V_SPEC,
                  _W_SPEC, _V_SPEC, _V_SPEC],
        out_specs=_ROW_SPEC,
        out_shape=jax.ShapeDtypeStruct((N_PAD, D), jnp.float32),
    )(parts, cnt, x, wl, bl.reshape(1, D), wr, g.reshape(1, D),
      b.reshape(1, D))


_MESH = plsc.VectorSubcoreMesh(core_axis_name="c", subcore_axis_name="s",
                               num_cores=NC, num_subcores=NS)


GRP = 8
NB = 2
N_GRP_TOTAL = E_PAD // (NC * NS * GRP * CH)
K0 = 15
K1 = 2 * N_GRP_TOTAL - K0


def _tile_layout(c, s):
    n_grp = jnp.where(c == 0, K0, K1)
    row_base = jnp.where(c == 0, s * (K0 * GRP),
                         16 * (K0 * GRP) + s * (K1 * GRP))
    return n_grp, row_base


def _edge_group(i, row_base, h_hbm, src2d_hbm, dst2d_hbm, src_g, dst_g, rows,
                agg_sh):
    row0 = pl.multiple_of(row_base + i * GRP, 8)
    pltpu.sync_copy(src2d_hbm.at[pl.ds(row0, GRP)], src_g)
    pltpu.sync_copy(dst2d_hbm.at[pl.ds(row0, GRP)], dst_g)

    for k in range(GRP):
        b = k % NB
        pltpu.sync_copy(h_hbm.at[src_g.at[k]], rows[b])
        pltpu.sync_copy(rows[b], agg_sh.at[dst_g.at[k]], add=True)


def _sc_body_cnt(h_hbm, src2d_hbm, dst2d_hbm, zrows_hbm, zcnt_hbm,
                 parts_out, cnt_out, src_g, dst_g, r0, r1,
                 cnt_v, agg_sh):
    rows = (r0, r1)
    c = lax.axis_index("c")
    s = lax.axis_index("s")
    wid = s * NC + c
    rbase = s * ROWS_PER_TILE
    pltpu.sync_copy(zrows_hbm.at[pl.ds(rbase, ROWS_PER_TILE)],
                    agg_sh.at[pl.ds(rbase, ROWS_PER_TILE)])
    pltpu.sync_copy(zcnt_hbm, cnt_v)
    plsc.subcore_barrier()

    ones16 = jnp.full((16,), 1.0, jnp.float32)
    n_grp, row_base = _tile_layout(c, s)

    @pl.loop(0, n_grp)
    def body(i):
        _edge_group(i, row_base, h_hbm, src2d_hbm, dst2d_hbm, src_g, dst_g,
                    rows, agg_sh)
        for k in range(GRP):
            for j in range(CH // 16):
                plsc.addupdate_scatter(
                    cnt_v, [dst_g[k, pl.ds(j * 16, 16)]], ones16)

    plsc.subcore_barrier()
    pltpu.sync_copy(agg_sh.at[pl.ds(rbase, ROWS_PER_TILE)],
                    parts_out.at[c, pl.ds(rbase, ROWS_PER_TILE)])
    pltpu.sync_copy(cnt_v, cnt_out.at[wid])


def _sc_body_nocnt(h_hbm, src2d_hbm, dst2d_hbm, zrows_hbm, parts_out, src_g,
                   dst_g, r0, r1, agg_sh):
    rows = (r0, r1)
    c = lax.axis_index("c")
    s = lax.axis_index("s")
    wid = s * NC + c
    rbase = s * ROWS_PER_TILE
    pltpu.sync_copy(zrows_hbm.at[pl.ds(rbase, ROWS_PER_TILE)],
                    agg_sh.at[pl.ds(rbase, ROWS_PER_TILE)])
    plsc.subcore_barrier()

    n_grp, row_base = _tile_layout(c, s)

    @pl.loop(0, n_grp)
    def body(i):
        _edge_group(i, row_base, h_hbm, src2d_hbm, dst2d_hbm, src_g, dst_g,
                    rows, agg_sh)

    plsc.subcore_barrier()
    pltpu.sync_copy(agg_sh.at[pl.ds(rbase, ROWS_PER_TILE)],
                    parts_out.at[c, pl.ds(rbase, ROWS_PER_TILE)])


_sc_edges_cnt = functools.partial(
    pl.kernel,
    _sc_body_cnt,
    out_type=(jax.ShapeDtypeStruct((NC, N_PAD, D), jnp.float32),
              jax.ShapeDtypeStruct((NW, N_PAD), jnp.float32)),
    mesh=_MESH,
    scratch_types=[
        pltpu.VMEM((GRP, CH), jnp.int32),
        pltpu.VMEM((GRP, CH), jnp.int32),
        pltpu.VMEM((CH, D), jnp.float32),
        pltpu.VMEM((CH, D), jnp.float32),
        pltpu.VMEM((N_PAD,), jnp.float32),
        pltpu.VMEM_SHARED((N_PAD, D), jnp.float32),
    ],
    compiler_params=pltpu.CompilerParams(needs_layout_passes=False),
)()

_sc_edges_nocnt = functools.partial(
    pl.kernel,
    _sc_body_nocnt,
    out_type=jax.ShapeDtypeStruct((NC, N_PAD, D), jnp.float32),
    mesh=_MESH,
    scratch_types=[
        pltpu.VMEM((GRP, CH), jnp.int32),
        pltpu.VMEM((GRP, CH), jnp.int32),
        pltpu.VMEM((CH, D), jnp.float32),
        pltpu.VMEM((CH, D), jnp.float32),
        pltpu.VMEM_SHARED((N_PAD, D), jnp.float32),
    ],
)()


@jax.jit
def kernel(x, edge_index, Wp0, bp0, Wl0, bl0, Wr0, g0, b0,
           Wp1, bp1, Wl1, bl1, Wr1, g1, b1):
    x_pad = jnp.pad(x, ((0, N_PAD - N), (0, 0)))
    src = jnp.pad(edge_index[0], (0, E_PAD - E))
    dst = jnp.pad(edge_index[1], (0, E_PAD - E), constant_values=N_PAD - 1)
    zrows = jnp.zeros((N_PAD, D), jnp.float32)
    zcnt = jnp.zeros((N_PAD,), jnp.float32)

    src2d = src.reshape(E_PAD // CH, CH)
    dst2d = dst.reshape(E_PAD // CH, CH)

    h0p = _tc_proj(x_pad, Wp0, bp0)
    parts0, cnt = _sc_edges_cnt(h0p, src2d, dst2d, zrows, zcnt)
    h0, h1p = _tc_mid(parts0, cnt, x_pad, Wl0, bl0, Wr0, g0, b0, Wp1, bp1)
    parts1 = _sc_edges_nocnt(h1p, src2d, dst2d, zrows)
    h1 = _tc_final(parts1, cnt, h0, Wl1, bl1, Wr1, g1, b1)
    return h1[:N]


# speedup vs baseline: 1.3772x; 1.0463x over previous
import functools

import jax
import jax.numpy as jnp
from jax import lax
from jax.experimental import pallas as pl
from jax.experimental.pallas import tpu as pltpu
from jax.experimental.pallas import tpu_sc as plsc

N = 10000
D = 128
E = 320000

NC = 2
NS = 16
NW = NC * NS

N_PAD = 10240
ROWS_PER_TILE = N_PAD // NS
CH = 128
E_PER_TILE = 10240
E_PAD = NW * E_PER_TILE
N_CH = E_PER_TILE // CH

BLK = 1024
GRID = N_PAD // BLK


def _dotT(a, w):
    return lax.dot_general(a, w, (((1,), (1,)), ((), ())),
                           precision=lax.Precision.HIGHEST,
                           preferred_element_type=jnp.float32)


def _proj_body(x_ref, w_ref, b_ref, o_ref):
    o_ref[...] = jnp.maximum(_dotT(x_ref[...], w_ref[...]) + b_ref[...], 0.0)


def _tc_proj(x, w, b):
    return pl.pallas_call(
        _proj_body,
        grid=(GRID,),
        in_specs=[
            pl.BlockSpec((BLK, D), lambda i: (i, 0)),
            pl.BlockSpec((D, D), lambda i: (0, 0)),
            pl.BlockSpec((1, D), lambda i: (0, 0)),
        ],
        out_specs=pl.BlockSpec((BLK, D), lambda i: (i, 0)),
        out_shape=jax.ShapeDtypeStruct((N_PAD, D), jnp.float32),
    )(x, w, b.reshape(1, D))


def _combine(parts_ref, cnt_ref, x_ref, wl_ref, bl_ref, wr_ref, g_ref, b_ref):
    p = parts_ref[0] + parts_ref[1]
    c = jnp.sum(cnt_ref[...], axis=0)[:, None]
    agg = p / jnp.maximum(c, 1.0)
    out = _dotT(agg, wl_ref[...]) + bl_ref[...] + _dotT(x_ref[...], wr_ref[...])
    mu = jnp.mean(out, axis=-1, keepdims=True)
    d = out - mu
    var = jnp.mean(d * d, axis=-1, keepdims=True)
    return d / jnp.sqrt(var + 1e-5) * g_ref[...] + b_ref[...]


def _mid_body(parts_ref, cnt_ref, x_ref, wl_ref, bl_ref, wr_ref, g_ref, b_ref,
              wp_ref, bp_ref, y_ref, h_ref):
    y = _combine(parts_ref, cnt_ref, x_ref, wl_ref, bl_ref, wr_ref, g_ref, b_ref)
    y_ref[...] = y
    h_ref[...] = jnp.maximum(_dotT(y, wp_ref[...]) + bp_ref[...], 0.0)


def _final_body(parts_ref, cnt_ref, x_ref, wl_ref, bl_ref, wr_ref, g_ref,
                b_ref, y_ref):
    y_ref[...] = _combine(parts_ref, cnt_ref, x_ref, wl_ref, bl_ref, wr_ref,
                          g_ref, b_ref)


_W_SPEC = pl.BlockSpec((D, D), lambda i: (0, 0))
_V_SPEC = pl.BlockSpec((1, D), lambda i: (0, 0))
_ROW_SPEC = pl.BlockSpec((BLK, D), lambda i: (i, 0))
_PARTS_SPEC = pl.BlockSpec((NC, BLK, D), lambda i: (0, i, 0))
_CNT_SPEC = pl.BlockSpec((NW, BLK), lambda i: (0, i))


def _tc_mid(parts, cnt, x, wl, bl, wr, g, b, wp, bp):
    return pl.pallas_call(
        _mid_body,
        grid=(GRID,),
        in_specs=[_PARTS_SPEC, _CNT_SPEC, _ROW_SPEC, _W_SPEC, _V_SPEC,
                  _W_SPEC, _V_SPEC, _V_SPEC, _W_SPEC, _V_SPEC],
        out_specs=(_ROW_SPEC, _ROW_SPEC),
        out_shape=(jax.ShapeDtypeStruct((N_PAD, D), jnp.float32),
                   jax.ShapeDtypeStruct((N_PAD, D), jnp.float32)),
    )(parts, cnt, x, wl, bl.reshape(1, D), wr, g.reshape(1, D),
      b.reshape(1, D), wp, bp.reshape(1, D))


def _tc_final(parts, cnt, x, wl, bl, wr, g, b):
    return pl.pallas_call(
        _final_body,
        grid=(GRID,),
        in_specs=[_PARTS_SPEC, _CNT_SPEC, _ROW_SPEC, _W_SPEC, _V_SPEC,
                  _W_SPEC, _V_SPEC, _V_SPEC],
        out_specs=_ROW_SPEC,
        out_shape=jax.ShapeDtypeStruct((N_PAD, D), jnp.float32),
    )(parts, cnt, x, wl, bl.reshape(1, D), wr, g.reshape(1, D),
      b.reshape(1, D))


_MESH = plsc.VectorSubcoreMesh(core_axis_name="c", subcore_axis_name="s",
                               num_cores=NC, num_subcores=NS)


GRP = 8
NB = 2
N_GRP_TOTAL = E_PAD // (NC * NS * GRP * CH)
K0 = 16
K1 = 2 * N_GRP_TOTAL - K0


def _tile_layout(c, s):
    n_grp = jnp.where(c == 0, K0, K1)
    row_base = jnp.where(c == 0, s * (K0 * GRP),
                         16 * (K0 * GRP) + s * (K1 * GRP))
    return n_grp, row_base


def _edge_group(i, row_base, h_hbm, src2d_hbm, dst2d_hbm, src_g, dst_g, rows,
                agg_sh):
    row0 = pl.multiple_of(row_base + i * GRP, 8)
    pltpu.sync_copy(src2d_hbm.at[pl.ds(row0, GRP)], src_g)
    pltpu.sync_copy(dst2d_hbm.at[pl.ds(row0, GRP)], dst_g)

    for k in range(GRP):
        b = k % NB
        pltpu.sync_copy(h_hbm.at[src_g.at[k]], rows[b])
        pltpu.sync_copy(rows[b], agg_sh.at[dst_g.at[k]], add=True)


def _sc_body_cnt(h_hbm, src2d_hbm, dst2d_hbm, zrows_hbm, zcnt_hbm,
                 parts_out, cnt_out, src_g, dst_g, r0, r1,
                 cnt_v, agg_sh):
    rows = (r0, r1)
    c = lax.axis_index("c")
    s = lax.axis_index("s")
    wid = s * NC + c
    rbase = s * ROWS_PER_TILE
    pltpu.sync_copy(zrows_hbm.at[pl.ds(rbase, ROWS_PER_TILE)],
                    agg_sh.at[pl.ds(rbase, ROWS_PER_TILE)])
    pltpu.sync_copy(zcnt_hbm, cnt_v)
    plsc.subcore_barrier()

    ones16 = jnp.full((16,), 1.0, jnp.float32)
    n_grp, row_base = _tile_layout(c, s)

    @pl.loop(0, n_grp)
    def body(i):
        _edge_group(i, row_base, h_hbm, src2d_hbm, dst2d_hbm, src_g, dst_g,
                    rows, agg_sh)
        for k in range(GRP):
            for j in range(CH // 16):
                plsc.addupdate_scatter(
                    cnt_v, [dst_g[k, pl.ds(j * 16, 16)]], ones16)

    plsc.subcore_barrier()
    pltpu.sync_copy(agg_sh.at[pl.ds(rbase, ROWS_PER_TILE)],
                    parts_out.at[c, pl.ds(rbase, ROWS_PER_TILE)])
    pltpu.sync_copy(cnt_v, cnt_out.at[wid])


def _sc_body_nocnt(h_hbm, src2d_hbm, dst2d_hbm, zrows_hbm, parts_out, src_g,
                   dst_g, r0, r1, agg_sh):
    rows = (r0, r1)
    c = lax.axis_index("c")
    s = lax.axis_index("s")
    wid = s * NC + c
    rbase = s * ROWS_PER_TILE
    pltpu.sync_copy(zrows_hbm.at[pl.ds(rbase, ROWS_PER_TILE)],
                    agg_sh.at[pl.ds(rbase, ROWS_PER_TILE)])
    plsc.subcore_barrier()

    n_grp, row_base = _tile_layout(c, s)

    @pl.loop(0, n_grp)
    def body(i):
        _edge_group(i, row_base, h_hbm, src2d_hbm, dst2d_hbm, src_g, dst_g,
                    rows, agg_sh)

    plsc.subcore_barrier()
    pltpu.sync_copy(agg_sh.at[pl.ds(rbase, ROWS_PER_TILE)],
                    parts_out.at[c, pl.ds(rbase, ROWS_PER_TILE)])


_sc_edges_cnt = functools.partial(
    pl.kernel,
    _sc_body_cnt,
    out_type=(jax.ShapeDtypeStruct((NC, N_PAD, D), jnp.float32),
              jax.ShapeDtypeStruct((NW, N_PAD), jnp.float32)),
    mesh=_MESH,
    scratch_types=[
        pltpu.VMEM((GRP, CH), jnp.int32),
        pltpu.VMEM((GRP, CH), jnp.int32),
        pltpu.VMEM((CH, D), jnp.float32),
        pltpu.VMEM((CH, D), jnp.float32),
        pltpu.VMEM((N_PAD,), jnp.float32),
        pltpu.VMEM_SHARED((N_PAD, D), jnp.float32),
    ],
    compiler_params=pltpu.CompilerParams(needs_layout_passes=False),
)()

_sc_edges_nocnt = functools.partial(
    pl.kernel,
    _sc_body_nocnt,
    out_type=jax.ShapeDtypeStruct((NC, N_PAD, D), jnp.float32),
    mesh=_MESH,
    scratch_types=[
        pltpu.VMEM((GRP, CH), jnp.int32),
        pltpu.VMEM((GRP, CH), jnp.int32),
        pltpu.VMEM((CH, D), jnp.float32),
        pltpu.VMEM((CH, D), jnp.float32),
        pltpu.VMEM_SHARED((N_PAD, D), jnp.float32),
    ],
)()


@jax.jit
def kernel(x, edge_index, Wp0, bp0, Wl0, bl0, Wr0, g0, b0,
           Wp1, bp1, Wl1, bl1, Wr1, g1, b1):
    x_pad = jnp.pad(x, ((0, N_PAD - N), (0, 0)))
    src = jnp.pad(edge_index[0], (0, E_PAD - E))
    dst = jnp.pad(edge_index[1], (0, E_PAD - E), constant_values=N_PAD - 1)
    zrows = jnp.zeros((N_PAD, D), jnp.float32)
    zcnt = jnp.zeros((N_PAD,), jnp.float32)

    src2d = src.reshape(E_PAD // CH, CH)
    dst2d = dst.reshape(E_PAD // CH, CH)

    h0p = _tc_proj(x_pad, Wp0, bp0)
    parts0, cnt = _sc_edges_cnt(h0p, src2d, dst2d, zrows, zcnt)
    h0, h1p = _tc_mid(parts0, cnt, x_pad, Wl0, bl0, Wr0, g0, b0, Wp1, bp1)
    parts1 = _sc_edges_nocnt(h1p, src2d, dst2d, zrows)
    h1 = _tc_final(parts1, cnt, h0, Wl1, bl1, Wr1, g1, b1)
    return h1[:N]


# speedup vs baseline: 1.4035x; 1.0191x over previous
import functools

import jax
import jax.numpy as jnp
from jax import lax
from jax.experimental import pallas as pl
from jax.experimental.pallas import tpu as pltpu
from jax.experimental.pallas import tpu_sc as plsc

N = 10000
D = 128
E = 320000

NC = 2
NS = 16
NW = NC * NS

N_PAD = 10240
ROWS_PER_TILE = N_PAD // NS
CH = 128
E_PER_TILE = 10240
E_PAD = NW * E_PER_TILE
N_CH = E_PER_TILE // CH

BLK = 1024
GRID = N_PAD // BLK


def _dotT(a, w):
    return lax.dot_general(a, w, (((1,), (1,)), ((), ())),
                           precision=lax.Precision.HIGHEST,
                           preferred_element_type=jnp.float32)


def _proj_body(x_ref, w_ref, b_ref, o_ref):
    o_ref[...] = jnp.maximum(_dotT(x_ref[...], w_ref[...]) + b_ref[...], 0.0)


def _tc_proj(x, w, b):
    return pl.pallas_call(
        _proj_body,
        grid=(GRID,),
        in_specs=[
            pl.BlockSpec((BLK, D), lambda i: (i, 0)),
            pl.BlockSpec((D, D), lambda i: (0, 0)),
            pl.BlockSpec((1, D), lambda i: (0, 0)),
        ],
        out_specs=pl.BlockSpec((BLK, D), lambda i: (i, 0)),
        out_shape=jax.ShapeDtypeStruct((N_PAD, D), jnp.float32),
    )(x, w, b.reshape(1, D))


def _combine(parts_ref, cnt_ref, x_ref, wl_ref, bl_ref, wr_ref, g_ref, b_ref):
    p = parts_ref[0] + parts_ref[1]
    c = jnp.sum(cnt_ref[...], axis=0)[:, None]
    agg = p / jnp.maximum(c, 1.0)
    out = _dotT(agg, wl_ref[...]) + bl_ref[...] + _dotT(x_ref[...], wr_ref[...])
    mu = jnp.mean(out, axis=-1, keepdims=True)
    d = out - mu
    var = jnp.mean(d * d, axis=-1, keepdims=True)
    return d / jnp.sqrt(var + 1e-5) * g_ref[...] + b_ref[...]


def _mid_body(parts_ref, cnt_ref, x_ref, wl_ref, bl_ref, wr_ref, g_ref, b_ref,
              wp_ref, bp_ref, y_ref, h_ref):
    y = _combine(parts_ref, cnt_ref, x_ref, wl_ref, bl_ref, wr_ref, g_ref, b_ref)
    y_ref[...] = y
    h_ref[...] = jnp.maximum(_dotT(y, wp_ref[...]) + bp_ref[...], 0.0)


def _final_body(parts_ref, cnt_ref, x_ref, wl_ref, bl_ref, wr_ref, g_ref,
                b_ref, y_ref):
    y_ref[...] = _combine(parts_ref, cnt_ref, x_ref, wl_ref, bl_ref, wr_ref,
                          g_ref, b_ref)


_W_SPEC = pl.BlockSpec((D, D), lambda i: (0, 0))
_V_SPEC = pl.BlockSpec((1, D), lambda i: (0, 0))
_ROW_SPEC = pl.BlockSpec((BLK, D), lambda i: (i, 0))
_PARTS_SPEC = pl.BlockSpec((NC, BLK, D), lambda i: (0, i, 0))
_CNT_SPEC = pl.BlockSpec((NW, BLK), lambda i: (0, i))


def _tc_mid(parts, cnt, x, wl, bl, wr, g, b, wp, bp):
    return pl.pallas_call(
        _mid_body,
        grid=(GRID,),
        in_specs=[_PARTS_SPEC, _CNT_SPEC, _ROW_SPEC, _W_SPEC, _V_SPEC,
                  _W_SPEC, _V_SPEC, _V_SPEC, _W_SPEC, _V_SPEC],
        out_specs=(_ROW_SPEC, _ROW_SPEC),
        out_shape=(jax.ShapeDtypeStruct((N_PAD, D), jnp.float32),
                   jax.ShapeDtypeStruct((N_PAD, D), jnp.float32)),
    )(parts, cnt, x, wl, bl.reshape(1, D), wr, g.reshape(1, D),
      b.reshape(1, D), wp, bp.reshape(1, D))


def _tc_final(parts, cnt, x, wl, bl, wr, g, b):
    return pl.pallas_call(
        _final_body,
        grid=(GRID,),
        in_specs=[_PARTS_SPEC, _CNT_SPEC, _ROW_SPEC, _W_SPEC, _V_SPEC,
                  _W_SPEC, _V_SPEC, _V_SPEC],
        out_specs=_ROW_SPEC,
        out_shape=jax.ShapeDtypeStruct((N_PAD, D), jnp.float32),
    )(parts, cnt, x, wl, bl.reshape(1, D), wr, g.reshape(1, D),
      b.reshape(1, D))


_MESH = plsc.VectorSubcoreMesh(core_axis_name="c", subcore_axis_name="s",
                               num_cores=NC, num_subcores=NS)


GRP = 8
NB = 2
N_GRP_TOTAL = E_PAD // (NC * NS * GRP * CH)
K0 = 18
K1 = 2 * N_GRP_TOTAL - K0


def _tile_layout(c, s):
    n_grp = jnp.where(c == 0, K0, K1)
    row_base = jnp.where(c == 0, s * (K0 * GRP),
                         16 * (K0 * GRP) + s * (K1 * GRP))
    return n_grp, row_base


def _edge_group(i, row_base, h_hbm, src2d_hbm, dst2d_hbm, src_g, dst_g, rows,
                agg_sh):
    row0 = pl.multiple_of(row_base + i * GRP, 8)
    pltpu.sync_copy(src2d_hbm.at[pl.ds(row0, GRP)], src_g)
    pltpu.sync_copy(dst2d_hbm.at[pl.ds(row0, GRP)], dst_g)

    for k in range(GRP):
        b = k % NB
        pltpu.sync_copy(h_hbm.at[src_g.at[k]], rows[b])
        pltpu.sync_copy(rows[b], agg_sh.at[dst_g.at[k]], add=True)


def _sc_body_cnt(h_hbm, src2d_hbm, dst2d_hbm, zrows_hbm, zcnt_hbm,
                 parts_out, cnt_out, src_g, dst_g, r0, r1,
                 cnt_v, agg_sh):
    rows = (r0, r1)
    c = lax.axis_index("c")
    s = lax.axis_index("s")
    wid = s * NC + c
    rbase = s * ROWS_PER_TILE
    pltpu.sync_copy(zrows_hbm.at[pl.ds(rbase, ROWS_PER_TILE)],
                    agg_sh.at[pl.ds(rbase, ROWS_PER_TILE)])
    pltpu.sync_copy(zcnt_hbm, cnt_v)
    plsc.subcore_barrier()

    ones16 = jnp.full((16,), 1.0, jnp.float32)
    n_grp, row_base = _tile_layout(c, s)

    @pl.loop(0, n_grp)
    def body(i):
        _edge_group(i, row_base, h_hbm, src2d_hbm, dst2d_hbm, src_g, dst_g,
                    rows, agg_sh)
        for k in range(GRP):
            for j in range(CH // 16):
                plsc.addupdate_scatter(
                    cnt_v, [dst_g[k, pl.ds(j * 16, 16)]], ones16)

    plsc.subcore_barrier()
    pltpu.sync_copy(agg_sh.at[pl.ds(rbase, ROWS_PER_TILE)],
                    parts_out.at[c, pl.ds(rbase, ROWS_PER_TILE)])
    pltpu.sync_copy(cnt_v, cnt_out.at[wid])


def _sc_body_nocnt(h_hbm, src2d_hbm, dst2d_hbm, zrows_hbm, parts_out, src_g,
                   dst_g, r0, r1, agg_sh):
    rows = (r0, r1)
    c = lax.axis_index("c")
    s = lax.axis_index("s")
    wid = s * NC + c
    rbase = s * ROWS_PER_TILE
    pltpu.sync_copy(zrows_hbm.at[pl.ds(rbase, ROWS_PER_TILE)],
                    agg_sh.at[pl.ds(rbase, ROWS_PER_TILE)])
    plsc.subcore_barrier()

    n_grp, row_base = _tile_layout(c, s)

    @pl.loop(0, n_grp)
    def body(i):
        _edge_group(i, row_base, h_hbm, src2d_hbm, dst2d_hbm, src_g, dst_g,
                    rows, agg_sh)

    plsc.subcore_barrier()
    pltpu.sync_copy(agg_sh.at[pl.ds(rbase, ROWS_PER_TILE)],
                    parts_out.at[c, pl.ds(rbase, ROWS_PER_TILE)])


_sc_edges_cnt = functools.partial(
    pl.kernel,
    _sc_body_cnt,
    out_type=(jax.ShapeDtypeStruct((NC, N_PAD, D), jnp.float32),
              jax.ShapeDtypeStruct((NW, N_PAD), jnp.float32)),
    mesh=_MESH,
    scratch_types=[
        pltpu.VMEM((GRP, CH), jnp.int32),
        pltpu.VMEM((GRP, CH), jnp.int32),
        pltpu.VMEM((CH, D), jnp.float32),
        pltpu.VMEM((CH, D), jnp.float32),
        pltpu.VMEM((N_PAD,), jnp.float32),
        pltpu.VMEM_SHARED((N_PAD, D), jnp.float32),
    ],
    compiler_params=pltpu.CompilerParams(needs_layout_passes=False),
)()

_sc_edges_nocnt = functools.partial(
    pl.kernel,
    _sc_body_nocnt,
    out_type=jax.ShapeDtypeStruct((NC, N_PAD, D), jnp.float32),
    mesh=_MESH,
    scratch_types=[
        pltpu.VMEM((GRP, CH), jnp.int32),
        pltpu.VMEM((GRP, CH), jnp.int32),
        pltpu.VMEM((CH, D), jnp.float32),
        pltpu.VMEM((CH, D), jnp.float32),
        pltpu.VMEM_SHARED((N_PAD, D), jnp.float32),
    ],
)()


@jax.jit
def kernel(x, edge_index, Wp0, bp0, Wl0, bl0, Wr0, g0, b0,
           Wp1, bp1, Wl1, bl1, Wr1, g1, b1):
    x_pad = jnp.pad(x, ((0, N_PAD - N), (0, 0)))
    src = jnp.pad(edge_index[0], (0, E_PAD - E))
    dst = jnp.pad(edge_index[1], (0, E_PAD - E), constant_values=N_PAD - 1)
    zrows = jnp.zeros((N_PAD, D), jnp.float32)
    zcnt = jnp.zeros((N_PAD,), jnp.float32)

    src2d = src.reshape(E_PAD // CH, CH)
    dst2d = dst.reshape(E_PAD // CH, CH)

    h0p = _tc_proj(x_pad, Wp0, bp0)
    parts0, cnt = _sc_edges_cnt(h0p, src2d, dst2d, zrows, zcnt)
    h0, h1p = _tc_mid(parts0, cnt, x_pad, Wl0, bl0, Wr0, g0, b0, Wp1, bp1)
    parts1 = _sc_edges_nocnt(h1p, src2d, dst2d, zrows)
    h1 = _tc_final(parts1, cnt, h0, Wl1, bl1, Wr1, g1, b1)
    return h1[:N]
